# Initial kernel scaffold; baseline (speedup 1.0000x reference)
import jax
import jax.numpy as jnp
from jax.experimental import pallas as pl


def kernel(msa, pair, R_in, T_in, xyz, state, idx, motif_mask, top_k, g_msa_ln, b_msa_ln, g_pair_ln, b_pair_ln, g_state_ln, b_state_ln, g_node_ln, b_node_ln, g_e1_ln, b_e1_ln, g_e2_ln, b_e2_ln, g_s0_ln, b_s0_ln, g_si_ln, b_si_ln, W_x, bb_x, W_e1, bb_e1, W_e2, bb_e2, W_msg, bb_msg, W_st, bb_st, W_off, bb_off, W_s0, bb_s0, W_si, bb_si, W_l1, bb_l1, W_l2, bb_l2, W_l3, bb_l3, W_l4, bb_l4, W_out, bb_out):
    raise NotImplementedError("write your pallas kernel here")


# speedup vs baseline: 3.5356x; 3.5356x over previous
import functools

import jax
import jax.numpy as jnp
from jax.experimental import pallas as pl

B, N, L = 1, 8, 512
D_MSA, D_PAIR, D_STATE = 256, 128, 16
L0_IN, L0_OUT, D_EDGE = 32, 16, 32
D_HID = 128
H_MSG = 64
TOPK = 64

BI = 64
BJ = 128
NI = L // BI
NJ = L // BJ


def _ln(x, g, b, eps=1e-5):
    m = jnp.mean(x, axis=-1, keepdims=True)
    v = jnp.var(x, axis=-1, keepdims=True)
    return (x - m) / jnp.sqrt(v + eps) * g + b


def _prep_kernel(ca_ref, caT_ref, msa0_ref, state_ref,
                 g_msa_ref, b_msa_ref, g_state_ref, b_state_ref,
                 wxa_ref, wxb_ref, bbx_ref, g_node_ref, b_node_ref,
                 dist_ref, maskf_ref, node_ref):
    ca = ca_ref[...]
    caT = caT_ref[...]
    dx = ca[:, 0:1] - caT[0:1, :]
    dy = ca[:, 1:2] - caT[1:2, :]
    dz = ca[:, 2:3] - caT[2:3, :]
    dist = jnp.sqrt(dx * dx + dy * dy + dz * dz + 1e-8)
    dist_ref[...] = dist

    bits = jax.lax.bitcast_convert_type(dist, jnp.int32)
    lo0 = jnp.min(bits, axis=1, keepdims=True)
    hi0 = jnp.max(bits, axis=1, keepdims=True)

    def body(_, carry):
        lo, hi = carry
        mid = lo + (hi - lo) // 2
        cnt = jnp.sum((bits <= mid).astype(jnp.float32), axis=1, keepdims=True)
        ge = cnt >= float(TOPK)
        return jnp.where(ge, lo, mid + 1), jnp.where(ge, mid, hi)

    lo, hi = jax.lax.fori_loop(0, 31, body, (lo0, hi0))
    v = lo
    mask_lt = bits < v
    cnt_lt = jnp.sum(mask_lt.astype(jnp.float32), axis=1, keepdims=True)
    tie = bits == v
    tie_f = tie.astype(jnp.float32)
    ii = jax.lax.broadcasted_iota(jnp.int32, (L, L), 0)
    jj = jax.lax.broadcasted_iota(jnp.int32, (L, L), 1)
    M = (ii <= jj).astype(jnp.float32)
    cs = jnp.dot(tie_f, M, preferred_element_type=jnp.float32)
    mask_tie = tie & (cs <= (float(TOPK) - cnt_lt))
    maskf_ref[...] = jnp.where(mask_lt | mask_tie, 1.0, 0.0)

    mln = _ln(msa0_ref[...], g_msa_ref[...], b_msa_ref[...])
    sln = _ln(state_ref[...], g_state_ref[...], b_state_ref[...])
    x = (jnp.dot(mln, wxa_ref[...], preferred_element_type=jnp.float32)
         + jnp.dot(sln, wxb_ref[...], preferred_element_type=jnp.float32)
         + bbx_ref[...])
    node_ref[...] = _ln(x, g_node_ref[...], b_node_ref[...])


def _pair_kernel(pair_ref, dist_ref, maskf_ref, nodei_ref, nodej_ref,
                 idxc_ref, idxr_ref,
                 g_pair_ref, b_pair_ref, we1_ref, bbe1_ref, g_e1_ref, b_e1_ref,
                 we2a_ref, we2b_ref, we2c_ref, bbe2_ref, g_e2_ref, b_e2_ref,
                 wmi_ref, wmj_ref, wme_ref, bbm_ref,
                 acc_ref):
    j = pl.program_id(1)

    p = pair_ref[0]
    pn = _ln(p, g_pair_ref[...], b_pair_ref[...])
    e1 = (jnp.dot(pn.reshape(BI * BJ, D_PAIR), we1_ref[...],
                  preferred_element_type=jnp.float32) + bbe1_ref[...])
    e1 = _ln(e1, g_e1_ref[...], b_e1_ref[...])

    d = dist_ref[...]
    kidx = jax.lax.broadcasted_iota(jnp.int32, (1, 1, 36), 2).astype(jnp.float32)
    centers = 2.0 + kidx * (20.0 / 35.0)
    sigma = 20.0 / 36.0
    rbf = jnp.exp(-(((d[..., None] - centers) / sigma) ** 2))

    off = idxr_ref[...] - idxc_ref[...]
    seqsep = jnp.sign(off) * jnp.log(jnp.abs(off) + 1.0)

    e2f = (jnp.dot(e1, we2a_ref[...], preferred_element_type=jnp.float32)
           + jnp.dot(rbf.reshape(BI * BJ, 36), we2b_ref[...],
                     preferred_element_type=jnp.float32)
           + bbe2_ref[...])
    e2 = e2f.reshape(BI, BJ, D_EDGE) + seqsep[..., None] * we2c_ref[...]
    e = _ln(e2, g_e2_ref[...], b_e2_ref[...]).reshape(BI * BJ, D_EDGE)

    mi = jnp.dot(nodei_ref[...], wmi_ref[...],
                 preferred_element_type=jnp.float32)
    mj = jnp.dot(nodej_ref[...], wmj_ref[...],
                 preferred_element_type=jnp.float32)
    me = jnp.dot(e, wme_ref[...],
                 preferred_element_type=jnp.float32).reshape(BI, BJ, H_MSG)
    msg = jax.nn.relu(me + mi[:, None, :] + mj[None, :, :] + bbm_ref[...])
    msg = msg * maskf_ref[...][..., None]
    partial = jnp.sum(msg, axis=1)

    @pl.when(j == 0)
    def _():
        acc_ref[...] = jnp.zeros_like(acc_ref)

    acc_ref[...] += partial


def _head_kernel(agg_ref, msa0_ref, r9_ref, tin_ref,
                 wst_ref, bbst_ref, woff_ref, bboff_ref,
                 g_s0_ref, b_s0_ref, g_si_ref, b_si_ref,
                 ws0_ref, bbs0_ref, wsi_ref, bbsi_ref,
                 wl1_ref, bl1_ref, wl2_ref, bl2_ref,
                 wl3_ref, bl3_ref, wl4_ref, bl4_ref,
                 wout_ref, bout_ref,
                 t_ref, ns_ref, alpha_ref):
    agg = agg_ref[...] * (1.0 / TOPK)
    ns = jnp.dot(agg, wst_ref[...],
                 preferred_element_type=jnp.float32) + bbst_ref[...]
    ns_ref[...] = ns
    off6 = jnp.dot(agg, woff_ref[...],
                   preferred_element_type=jnp.float32) + bboff_ref[...]
    delT = off6[:, 0:3] / 10.0
    r9 = r9_ref[...]
    t0 = jnp.sum(r9[:, 0:3] * delT, axis=1, keepdims=True)
    t1 = jnp.sum(r9[:, 3:6] * delT, axis=1, keepdims=True)
    t2 = jnp.sum(r9[:, 6:9] * delT, axis=1, keepdims=True)
    t_ref[...] = jnp.concatenate([t0, t1, t2], axis=1) + tin_ref[...]

    s0 = _ln(msa0_ref[...], g_s0_ref[...], b_s0_ref[...])
    si_in = _ln(ns, g_si_ref[...], b_si_ref[...])
    si = (jnp.dot(s0, ws0_ref[...], preferred_element_type=jnp.float32)
          + bbs0_ref[...]
          + jnp.dot(si_in, wsi_ref[...], preferred_element_type=jnp.float32)
          + bbsi_ref[...])
    h = jax.nn.relu(jnp.dot(jax.nn.relu(si), wl1_ref[...],
                            preferred_element_type=jnp.float32) + bl1_ref[...])
    si = si + jnp.dot(h, wl2_ref[...],
                      preferred_element_type=jnp.float32) + bl2_ref[...]
    h = jax.nn.relu(jnp.dot(jax.nn.relu(si), wl3_ref[...],
                            preferred_element_type=jnp.float32) + bl3_ref[...])
    si = si + jnp.dot(h, wl4_ref[...],
                      preferred_element_type=jnp.float32) + bl4_ref[...]
    alpha_ref[...] = (jnp.dot(jax.nn.relu(si), wout_ref[...],
                              preferred_element_type=jnp.float32)
                      + bout_ref[...])


def kernel(msa, pair, R_in, T_in, xyz, state, idx, motif_mask, top_k, g_msa_ln, b_msa_ln, g_pair_ln, b_pair_ln, g_state_ln, b_state_ln, g_node_ln, b_node_ln, g_e1_ln, b_e1_ln, g_e2_ln, b_e2_ln, g_s0_ln, b_s0_ln, g_si_ln, b_si_ln, W_x, bb_x, W_e1, bb_e1, W_e2, bb_e2, W_msg, bb_msg, W_st, bb_st, W_off, bb_off, W_s0, bb_s0, W_si, bb_si, W_l1, bb_l1, W_l2, bb_l2, W_l3, bb_l3, W_l4, bb_l4, W_out, bb_out):
    f32 = jnp.float32
    msa0 = msa[0, 0]
    ca = xyz[0, :, 1, :]
    caT = jnp.transpose(ca)
    state0 = state[0]
    idxf = idx[0].astype(f32)
    idx_col = idxf[:, None]
    idx_row = idxf[None, :]
    r9 = R_in[0].reshape(L, 9)
    tin = T_in[0]

    row = lambda a: a.reshape(1, -1)

    dist, maskf, node = pl.pallas_call(
        _prep_kernel,
        out_shape=[
            jax.ShapeDtypeStruct((L, L), f32),
            jax.ShapeDtypeStruct((L, L), f32),
            jax.ShapeDtypeStruct((L, L0_IN), f32),
        ],
    )(ca, caT, msa0, state0,
      row(g_msa_ln), row(b_msa_ln), row(g_state_ln), row(b_state_ln),
      W_x[:D_MSA], W_x[D_MSA:], row(bb_x), row(g_node_ln), row(b_node_ln))

    wspec = lambda shp: pl.BlockSpec(shp, lambda i, j: (0,) * len(shp))
    agg = pl.pallas_call(
        _pair_kernel,
        grid=(NI, NJ),
        in_specs=[
            pl.BlockSpec((1, BI, BJ, D_PAIR), lambda i, j: (0, i, j, 0)),
            pl.BlockSpec((BI, BJ), lambda i, j: (i, j)),
            pl.BlockSpec((BI, BJ), lambda i, j: (i, j)),
            pl.BlockSpec((BI, L0_IN), lambda i, j: (i, 0)),
            pl.BlockSpec((BJ, L0_IN), lambda i, j: (j, 0)),
            pl.BlockSpec((BI, 1), lambda i, j: (i, 0)),
            pl.BlockSpec((1, BJ), lambda i, j: (0, j)),
            wspec((1, D_PAIR)), wspec((1, D_PAIR)),
            wspec((D_PAIR, D_EDGE)), wspec((1, D_EDGE)),
            wspec((1, D_EDGE)), wspec((1, D_EDGE)),
            wspec((D_EDGE, D_EDGE)), wspec((36, D_EDGE)), wspec((1, D_EDGE)),
            wspec((1, D_EDGE)), wspec((1, D_EDGE)), wspec((1, D_EDGE)),
            wspec((L0_IN, H_MSG)), wspec((L0_IN, H_MSG)),
            wspec((D_EDGE, H_MSG)), wspec((1, H_MSG)),
        ],
        out_specs=pl.BlockSpec((BI, H_MSG), lambda i, j: (i, 0)),
        out_shape=jax.ShapeDtypeStruct((L, H_MSG), f32),
    )(pair, dist, maskf, node, node, idx_col, idx_row,
      row(g_pair_ln), row(b_pair_ln), W_e1, row(bb_e1),
      row(g_e1_ln), row(b_e1_ln),
      W_e2[:D_EDGE], W_e2[D_EDGE:D_EDGE + 36], W_e2[D_EDGE + 36:],
      row(bb_e2), row(g_e2_ln), row(b_e2_ln),
      W_msg[:L0_IN], W_msg[L0_IN:2 * L0_IN], W_msg[2 * L0_IN:], row(bb_msg))

    T, new_state, alpha = pl.pallas_call(
        _head_kernel,
        out_shape=[
            jax.ShapeDtypeStruct((L, 3), f32),
            jax.ShapeDtypeStruct((L, L0_OUT), f32),
            jax.ShapeDtypeStruct((L, 20), f32),
        ],
    )(agg, msa0, r9, tin,
      W_st, row(bb_st), W_off, row(bb_off),
      row(g_s0_ln), row(b_s0_ln), row(g_si_ln), row(b_si_ln),
      W_s0, row(bb_s0), W_si, row(bb_si),
      W_l1, row(bb_l1), W_l2, row(bb_l2),
      W_l3, row(bb_l3), W_l4, row(bb_l4),
      W_out, row(bb_out))

    return (R_in, T[None], new_state[None], alpha.reshape(1, L, 10, 2))
